# baseline (speedup 1.0000x reference)
import functools

import jax
import jax.numpy as jnp
from jax import lax
from jax.experimental import pallas as pl
from jax.experimental.pallas import tpu as pltpu
from jax.experimental.pallas import tpu_sc as plsc

NC = 2
NS = 16
NW = NC * NS
LANES = 16
BLANES = 32

DEGREE = 32
EMBED_DIM = 128
BWORDS = EMBED_DIM // BLANES

NBUF = 4
NACC = 8

CH = 64
CNB = 4


def _make_convert(rows):
  mesh = plsc.VectorSubcoreMesh(core_axis_name="c", subcore_axis_name="s")
  lo = rows // NW
  extra = rows - lo * NW
  trips = (lo + 1 + CH - 1) // CH
  trips = ((trips + CNB - 1) // CNB) * CNB

  @functools.partial(
      pl.kernel,
      out_type=jax.ShapeDtypeStruct((rows, EMBED_DIM), jnp.bfloat16),
      mesh=mesh,
      compiler_params=pltpu.CompilerParams(use_tc_tiling_on_sc=False,
                                           needs_layout_passes=False),
      scratch_types=[
          pltpu.VMEM((CNB, CH, EMBED_DIM), jnp.float32),
          pltpu.VMEM((CNB, CH, EMBED_DIM), jnp.bfloat16),
          pltpu.SemaphoreType.DMA((CNB,)),
          pltpu.SemaphoreType.DMA((CNB,)),
      ],
  )
  def conv(tab_hbm, out_hbm, in_v, out_v, isems, osems):
    w = lax.axis_index("s") * NC + lax.axis_index("c")
    base = w * lo + jnp.minimum(w, extra)
    cnt = lo + jnp.where(w < extra, 1, 0)

    def start_of(ci):
      return base + jnp.minimum(ci * CH, cnt - CH)

    for b in range(CNB):
      pltpu.async_copy(tab_hbm.at[pl.ds(start_of(b), CH)], in_v.at[b],
                       isems.at[b])

    def body(i, carry):
      for b in range(CNB):
        ci = i * CNB + b
        pltpu.make_async_copy(tab_hbm.at[pl.ds(start_of(ci), CH)],
                              in_v.at[b], isems.at[b]).wait()

        @pl.when(ci >= CNB)
        def _():
          pltpu.make_async_copy(
              out_v.at[b], out_hbm.at[pl.ds(start_of(ci - CNB), CH)],
              osems.at[b]).wait()

        @plsc.parallel_loop(0, CH, unroll=4)
        def _(r):
          for c in range(BWORDS):
            a = in_v[b, r, pl.ds(c * BLANES, LANES)]
            bb = in_v[b, r, pl.ds(c * BLANES + LANES, LANES)]
            out_v[b, r, pl.ds(c * BLANES, BLANES)] = plsc.pack(
                a, bb, format=plsc.PackFormat.INTERLEAVED)
        pltpu.async_copy(out_v.at[b], out_hbm.at[pl.ds(start_of(ci), CH)],
                         osems.at[b])

        ci2 = ci + CNB

        @pl.when(ci2 < trips)
        def _():
          pltpu.async_copy(tab_hbm.at[pl.ds(start_of(ci2), CH)], in_v.at[b],
                           isems.at[b])
      return carry

    lax.fori_loop(0, trips // CNB, body, 0)
    for b in range(CNB):
      pltpu.make_async_copy(
          out_v.at[b], out_hbm.at[pl.ds(start_of(trips - CNB + b), CH)],
          osems.at[b]).wait()

  return conv


def _make_gather(b, b_pad, rows):
  bpw = b_pad // NW
  tail = b - (NW - 1) * bpw
  mesh = plsc.VectorSubcoreMesh(core_axis_name="c", subcore_axis_name="s")

  @functools.partial(
      pl.kernel,
      out_type=jax.ShapeDtypeStruct((b, EMBED_DIM), jnp.float32),
      mesh=mesh,
      compiler_params=pltpu.CompilerParams(use_tc_tiling_on_sc=False,
                                           needs_layout_passes=False),
      scratch_types=[
          pltpu.VMEM((bpw,), jnp.int32),
          pltpu.VMEM((bpw, DEGREE), jnp.int32),
          pltpu.VMEM((NBUF, DEGREE, EMBED_DIM), jnp.bfloat16),
          pltpu.VMEM((bpw, EMBED_DIM), jnp.float32),
          pltpu.SemaphoreType.DMA((NBUF,)),
      ],
  )
  def k(nodes_hbm, nhe_hbm, table_hbm, out_hbm,
        nodes_v, he_ids_v, emb_v, out_v, sems):
    wid = lax.axis_index("s") * NC + lax.axis_index("c")
    base = wid * bpw
    pltpu.sync_copy(nodes_hbm.at[pl.ds(base, bpw)], nodes_v)
    for c in range(bpw // 64):
      pltpu.async_copy(
          nhe_hbm.at[nodes_v.at[pl.ds(c * 64, 64)]],
          he_ids_v.at[pl.ds(c * 64, 64)], sems.at[0])
    for c in range(bpw // 64):
      pltpu.make_async_copy(
          nhe_hbm.at[nodes_v.at[pl.ds(c * 64, 64)]],
          he_ids_v.at[pl.ds(c * 64, 64)], sems.at[0]).wait()

    for b in range(NBUF):
      pltpu.async_copy(table_hbm.at[he_ids_v.at[b]], emb_v.at[b], sems.at[b])

    def group_body(g, carry):
      for b in range(NBUF):
        n = g * NBUF + b
        pltpu.make_async_copy(
            table_hbm.at[he_ids_v.at[n]], emb_v.at[b], sems.at[b]).wait()
        for d in range(BWORDS):
          sl = pl.ds(d * BLANES, BLANES)
          accs = [emb_v[b, a, sl] for a in range(NACC)]
          for r in range(NACC, DEGREE):
            a = r % NACC
            accs[a] = jnp.maximum(accs[a], emb_v[b, r, sl])
          while len(accs) > 1:
            accs = [jnp.maximum(accs[2 * i], accs[2 * i + 1])
                    for i in range(len(accs) // 2)]
          acc = accs[0]
          ua, ub = plsc.unpack(acc, format=plsc.PackFormat.INTERLEAVED)
          out_v[n, pl.ds(d * BLANES, LANES)] = ua
          out_v[n, pl.ds(d * BLANES + LANES, LANES)] = ub
        n2 = n + NBUF

        @pl.when(n2 < bpw)
        def _():
          pltpu.async_copy(table_hbm.at[he_ids_v.at[n2]], emb_v.at[b],
                           sems.at[b])
      return carry

    lax.fori_loop(0, bpw // NBUF, group_body, 0)

    @pl.when(wid < NW - 1)
    def _():
      pltpu.sync_copy(out_v, out_hbm.at[pl.ds(base, bpw)])

    @pl.when(wid == NW - 1)
    def _():
      pltpu.sync_copy(out_v.at[pl.ds(0, tail)], out_hbm.at[pl.ds(base, tail)])

  return k


@jax.jit
def kernel(nodes, node_hyperedge_ids, hyperedge_table):
  b = nodes.shape[0]
  rows = hyperedge_table.shape[0]
  b_pad = ((b + 8 * NW - 1) // (8 * NW)) * (8 * NW)
  nodes_p = jnp.concatenate(
      [nodes, jnp.zeros((b_pad - b,), jnp.int32)]) if b_pad != b else nodes
  table_bf16 = _make_convert(rows)(hyperedge_table)
  return _make_gather(b, b_pad, rows)(nodes_p, node_hyperedge_ids, table_bf16)


# speedup vs baseline: 1.2144x; 1.2144x over previous
import functools

import jax
import jax.numpy as jnp
from jax import lax
from jax.experimental import pallas as pl
from jax.experimental.pallas import tpu as pltpu
from jax.experimental.pallas import tpu_sc as plsc

NC = 2
NS = 16
NW = NC * NS
LANES = 16
BLANES = 32

DEGREE = 32
EMBED_DIM = 128
BWORDS = EMBED_DIM // BLANES

NBUF = 8
NACC = 8

CH = 64
CNB = 4


def _make_convert(rows):
  mesh = plsc.VectorSubcoreMesh(core_axis_name="c", subcore_axis_name="s")
  lo = rows // NW
  extra = rows - lo * NW
  trips = (lo + 1 + CH - 1) // CH
  trips = ((trips + CNB - 1) // CNB) * CNB

  @functools.partial(
      pl.kernel,
      out_type=jax.ShapeDtypeStruct((rows, EMBED_DIM), jnp.bfloat16),
      mesh=mesh,
      compiler_params=pltpu.CompilerParams(use_tc_tiling_on_sc=False,
                                           needs_layout_passes=False),
      scratch_types=[
          pltpu.VMEM((CNB, CH, EMBED_DIM), jnp.float32),
          pltpu.VMEM((CNB, CH, EMBED_DIM), jnp.bfloat16),
          pltpu.SemaphoreType.DMA((CNB,)),
          pltpu.SemaphoreType.DMA((CNB,)),
      ],
  )
  def conv(tab_hbm, out_hbm, in_v, out_v, isems, osems):
    w = lax.axis_index("s") * NC + lax.axis_index("c")
    base = w * lo + jnp.minimum(w, extra)
    cnt = lo + jnp.where(w < extra, 1, 0)

    def start_of(ci):
      return base + jnp.minimum(ci * CH, cnt - CH)

    for b in range(CNB):
      pltpu.async_copy(tab_hbm.at[pl.ds(start_of(b), CH)], in_v.at[b],
                       isems.at[b])

    def body(i, carry):
      for b in range(CNB):
        ci = i * CNB + b
        pltpu.make_async_copy(tab_hbm.at[pl.ds(start_of(ci), CH)],
                              in_v.at[b], isems.at[b]).wait()

        @pl.when(ci >= CNB)
        def _():
          pltpu.make_async_copy(
              out_v.at[b], out_hbm.at[pl.ds(start_of(ci - CNB), CH)],
              osems.at[b]).wait()

        @plsc.parallel_loop(0, CH, unroll=8)
        def _(r):
          for c in range(BWORDS):
            a = in_v[b, r, pl.ds(c * BLANES, LANES)]
            bb = in_v[b, r, pl.ds(c * BLANES + LANES, LANES)]
            out_v[b, r, pl.ds(c * BLANES, BLANES)] = plsc.pack(
                a, bb, format=plsc.PackFormat.INTERLEAVED)
        pltpu.async_copy(out_v.at[b], out_hbm.at[pl.ds(start_of(ci), CH)],
                         osems.at[b])

        ci2 = ci + CNB

        @pl.when(ci2 < trips)
        def _():
          pltpu.async_copy(tab_hbm.at[pl.ds(start_of(ci2), CH)], in_v.at[b],
                           isems.at[b])
      return carry

    lax.fori_loop(0, trips // CNB, body, 0)
    for b in range(CNB):
      pltpu.make_async_copy(
          out_v.at[b], out_hbm.at[pl.ds(start_of(trips - CNB + b), CH)],
          osems.at[b]).wait()

  return conv


def _make_gather(b, b_pad, rows):
  bpw = b_pad // NW
  tail = b - (NW - 1) * bpw
  mesh = plsc.VectorSubcoreMesh(core_axis_name="c", subcore_axis_name="s")

  @functools.partial(
      pl.kernel,
      out_type=jax.ShapeDtypeStruct((b, EMBED_DIM), jnp.float32),
      mesh=mesh,
      compiler_params=pltpu.CompilerParams(use_tc_tiling_on_sc=False,
                                           needs_layout_passes=False),
      scratch_types=[
          pltpu.VMEM((bpw,), jnp.int32),
          pltpu.VMEM((bpw, DEGREE), jnp.int32),
          pltpu.VMEM((NBUF, DEGREE, EMBED_DIM), jnp.bfloat16),
          pltpu.VMEM((bpw, EMBED_DIM), jnp.float32),
          pltpu.SemaphoreType.DMA((NBUF,)),
      ],
  )
  def k(nodes_hbm, nhe_hbm, table_hbm, out_hbm,
        nodes_v, he_ids_v, emb_v, out_v, sems):
    wid = lax.axis_index("s") * NC + lax.axis_index("c")
    base = wid * bpw
    pltpu.sync_copy(nodes_hbm.at[pl.ds(base, bpw)], nodes_v)
    for c in range(bpw // 64):
      pltpu.async_copy(
          nhe_hbm.at[nodes_v.at[pl.ds(c * 64, 64)]],
          he_ids_v.at[pl.ds(c * 64, 64)], sems.at[0])
    for c in range(bpw // 64):
      pltpu.make_async_copy(
          nhe_hbm.at[nodes_v.at[pl.ds(c * 64, 64)]],
          he_ids_v.at[pl.ds(c * 64, 64)], sems.at[0]).wait()

    for b in range(NBUF):
      pltpu.async_copy(table_hbm.at[he_ids_v.at[b]], emb_v.at[b], sems.at[b])

    def group_body(g, carry):
      for b in range(NBUF):
        n = g * NBUF + b
        pltpu.make_async_copy(
            table_hbm.at[he_ids_v.at[n]], emb_v.at[b], sems.at[b]).wait()
        for d in range(BWORDS):
          sl = pl.ds(d * BLANES, BLANES)
          accs = [emb_v[b, a, sl] for a in range(NACC)]
          for r in range(NACC, DEGREE):
            a = r % NACC
            accs[a] = jnp.maximum(accs[a], emb_v[b, r, sl])
          while len(accs) > 1:
            accs = [jnp.maximum(accs[2 * i], accs[2 * i + 1])
                    for i in range(len(accs) // 2)]
          acc = accs[0]
          ua, ub = plsc.unpack(acc, format=plsc.PackFormat.INTERLEAVED)
          out_v[n, pl.ds(d * BLANES, LANES)] = ua
          out_v[n, pl.ds(d * BLANES + LANES, LANES)] = ub
        n2 = n + NBUF

        @pl.when(n2 < bpw)
        def _():
          pltpu.async_copy(table_hbm.at[he_ids_v.at[n2]], emb_v.at[b],
                           sems.at[b])
      return carry

    lax.fori_loop(0, bpw // NBUF, group_body, 0)

    @pl.when(wid < NW - 1)
    def _():
      pltpu.sync_copy(out_v, out_hbm.at[pl.ds(base, bpw)])

    @pl.when(wid == NW - 1)
    def _():
      pltpu.sync_copy(out_v.at[pl.ds(0, tail)], out_hbm.at[pl.ds(base, tail)])

  return k


@jax.jit
def kernel(nodes, node_hyperedge_ids, hyperedge_table):
  b = nodes.shape[0]
  rows = hyperedge_table.shape[0]
  b_pad = ((b + 8 * NW - 1) // (8 * NW)) * (8 * NW)
  nodes_p = jnp.concatenate(
      [nodes, jnp.zeros((b_pad - b,), jnp.int32)]) if b_pad != b else nodes
  table_bf16 = _make_convert(rows)(hyperedge_table)
  return _make_gather(b, b_pad, rows)(nodes_p, node_hyperedge_ids, table_bf16)


# speedup vs baseline: 1.2497x; 1.0291x over previous
import functools

import jax
import jax.numpy as jnp
from jax import lax
from jax.experimental import pallas as pl
from jax.experimental.pallas import tpu as pltpu
from jax.experimental.pallas import tpu_sc as plsc

NC = 2
NS = 16
NW = NC * NS
LANES = 16
BLANES = 32

DEGREE = 32
EMBED_DIM = 128
BWORDS = EMBED_DIM // BLANES

NBUF = 8
NACC = 4

CH = 64
CNB = 4


def _make_convert(rows):
  mesh = plsc.VectorSubcoreMesh(core_axis_name="c", subcore_axis_name="s")
  lo = rows // NW
  extra = rows - lo * NW
  trips = (lo + 1 + CH - 1) // CH
  trips = ((trips + CNB - 1) // CNB) * CNB

  @functools.partial(
      pl.kernel,
      out_type=jax.ShapeDtypeStruct((rows, EMBED_DIM), jnp.bfloat16),
      mesh=mesh,
      compiler_params=pltpu.CompilerParams(use_tc_tiling_on_sc=False,
                                           needs_layout_passes=False),
      scratch_types=[
          pltpu.VMEM((CNB, CH, EMBED_DIM), jnp.float32),
          pltpu.VMEM((CNB, CH, EMBED_DIM), jnp.bfloat16),
          pltpu.SemaphoreType.DMA((CNB,)),
          pltpu.SemaphoreType.DMA((CNB,)),
      ],
  )
  def conv(tab_hbm, out_hbm, in_v, out_v, isems, osems):
    w = lax.axis_index("s") * NC + lax.axis_index("c")
    base = w * lo + jnp.minimum(w, extra)
    cnt = lo + jnp.where(w < extra, 1, 0)

    def start_of(ci):
      return base + jnp.minimum(ci * CH, cnt - CH)

    for b in range(CNB):
      pltpu.async_copy(tab_hbm.at[pl.ds(start_of(b), CH)], in_v.at[b],
                       isems.at[b])

    def body(i, carry):
      for b in range(CNB):
        ci = i * CNB + b
        pltpu.make_async_copy(tab_hbm.at[pl.ds(start_of(ci), CH)],
                              in_v.at[b], isems.at[b]).wait()

        @pl.when(ci >= CNB)
        def _():
          pltpu.make_async_copy(
              out_v.at[b], out_hbm.at[pl.ds(start_of(ci - CNB), CH)],
              osems.at[b]).wait()

        @plsc.parallel_loop(0, CH, unroll=4)
        def _(r):
          for c in range(BWORDS):
            a = in_v[b, r, pl.ds(c * BLANES, LANES)]
            bb = in_v[b, r, pl.ds(c * BLANES + LANES, LANES)]
            out_v[b, r, pl.ds(c * BLANES, BLANES)] = plsc.pack(
                a, bb, format=plsc.PackFormat.INTERLEAVED)
        pltpu.async_copy(out_v.at[b], out_hbm.at[pl.ds(start_of(ci), CH)],
                         osems.at[b])

        ci2 = ci + CNB

        @pl.when(ci2 < trips)
        def _():
          pltpu.async_copy(tab_hbm.at[pl.ds(start_of(ci2), CH)], in_v.at[b],
                           isems.at[b])
      return carry

    lax.fori_loop(0, trips // CNB, body, 0)
    for b in range(CNB):
      pltpu.make_async_copy(
          out_v.at[b], out_hbm.at[pl.ds(start_of(trips - CNB + b), CH)],
          osems.at[b]).wait()

  return conv


def _make_gather(b, b_pad, rows):
  bpw = b_pad // NW
  tail = b - (NW - 1) * bpw
  mesh = plsc.VectorSubcoreMesh(core_axis_name="c", subcore_axis_name="s")

  @functools.partial(
      pl.kernel,
      out_type=jax.ShapeDtypeStruct((b, EMBED_DIM), jnp.float32),
      mesh=mesh,
      compiler_params=pltpu.CompilerParams(use_tc_tiling_on_sc=False,
                                           needs_layout_passes=False),
      scratch_types=[
          pltpu.VMEM((bpw,), jnp.int32),
          pltpu.VMEM((bpw, DEGREE), jnp.int32),
          pltpu.VMEM((NBUF, DEGREE, EMBED_DIM), jnp.bfloat16),
          pltpu.VMEM((bpw, EMBED_DIM), jnp.float32),
          pltpu.SemaphoreType.DMA((NBUF,)),
      ],
  )
  def k(nodes_hbm, nhe_hbm, table_hbm, out_hbm,
        nodes_v, he_ids_v, emb_v, out_v, sems):
    wid = lax.axis_index("s") * NC + lax.axis_index("c")
    base = wid * bpw
    pltpu.sync_copy(nodes_hbm.at[pl.ds(base, bpw)], nodes_v)
    for c in range(bpw // 64):
      pltpu.async_copy(
          nhe_hbm.at[nodes_v.at[pl.ds(c * 64, 64)]],
          he_ids_v.at[pl.ds(c * 64, 64)], sems.at[0])
    for c in range(bpw // 64):
      pltpu.make_async_copy(
          nhe_hbm.at[nodes_v.at[pl.ds(c * 64, 64)]],
          he_ids_v.at[pl.ds(c * 64, 64)], sems.at[0]).wait()

    for b in range(NBUF):
      pltpu.async_copy(table_hbm.at[he_ids_v.at[b]], emb_v.at[b], sems.at[b])

    def group_body(g, carry):
      for b in range(NBUF):
        n = g * NBUF + b
        pltpu.make_async_copy(
            table_hbm.at[he_ids_v.at[n]], emb_v.at[b], sems.at[b]).wait()
        for d in range(BWORDS):
          sl = pl.ds(d * BLANES, BLANES)
          accs = [emb_v[b, a, sl] for a in range(NACC)]
          for r in range(NACC, DEGREE):
            a = r % NACC
            accs[a] = jnp.maximum(accs[a], emb_v[b, r, sl])
          while len(accs) > 1:
            accs = [jnp.maximum(accs[2 * i], accs[2 * i + 1])
                    for i in range(len(accs) // 2)]
          acc = accs[0]
          ua, ub = plsc.unpack(acc, format=plsc.PackFormat.INTERLEAVED)
          out_v[n, pl.ds(d * BLANES, LANES)] = ua
          out_v[n, pl.ds(d * BLANES + LANES, LANES)] = ub
        n2 = n + NBUF

        @pl.when(n2 < bpw)
        def _():
          pltpu.async_copy(table_hbm.at[he_ids_v.at[n2]], emb_v.at[b],
                           sems.at[b])
      return carry

    lax.fori_loop(0, bpw // NBUF, group_body, 0)

    @pl.when(wid < NW - 1)
    def _():
      pltpu.sync_copy(out_v, out_hbm.at[pl.ds(base, bpw)])

    @pl.when(wid == NW - 1)
    def _():
      pltpu.sync_copy(out_v.at[pl.ds(0, tail)], out_hbm.at[pl.ds(base, tail)])

  return k


@jax.jit
def kernel(nodes, node_hyperedge_ids, hyperedge_table):
  b = nodes.shape[0]
  rows = hyperedge_table.shape[0]
  b_pad = ((b + 8 * NW - 1) // (8 * NW)) * (8 * NW)
  nodes_p = jnp.concatenate(
      [nodes, jnp.zeros((b_pad - b,), jnp.int32)]) if b_pad != b else nodes
  table_bf16 = _make_convert(rows)(hyperedge_table)
  return _make_gather(b, b_pad, rows)(nodes_p, node_hyperedge_ids, table_bf16)
